# baseline (speedup 1.0000x reference)
import functools

import jax
import jax.numpy as jnp
from jax import lax
from jax.experimental import pallas as pl
from jax.experimental.pallas import tpu as pltpu
from jax.experimental.pallas import tpu_sc as plsc

_BLK = 4096
_LANES = 16


def _scan_body(x_ref, y_ref, z_ref, rx_ref, ry_ref, rz_ref, vals_ref,
               *, blk, n_points):
    b = pl.program_id(0)
    rem = n_points - b * blk
    ok = lax.broadcasted_iota(jnp.int32, (1, blk), 1) < rem
    x = jnp.where(ok, x_ref[...].reshape(1, blk), 1e9)
    y = jnp.where(ok, y_ref[...].reshape(1, blk), 1e9)
    z = jnp.where(ok, z_ref[...].reshape(1, blk), 1e9)
    dx = x - rx_ref[...]
    dy = y - ry_ref[...]
    dz = z - rz_ref[...]
    d2 = (dx * dx + dy * dy) + dz * dz
    vals_ref[...] = jnp.min(d2, axis=1).reshape(1, 1, -1)


def _stage_a(xf, yf, zf, rx, ry, rz, num_blocks, blk, n_points):
    nb = rx.shape[0]
    coord_spec = pl.BlockSpec((blk,), lambda b: (b,))
    recv_spec = pl.BlockSpec((nb, 1), lambda b: (0, 0))
    return pl.pallas_call(
        functools.partial(_scan_body, blk=blk, n_points=n_points),
        grid=(num_blocks,),
        in_specs=[coord_spec, coord_spec, coord_spec,
                  recv_spec, recv_spec, recv_spec],
        out_specs=pl.BlockSpec((1, 1, nb), lambda b: (b, 0, 0)),
        out_shape=jax.ShapeDtypeStruct((num_blocks, 1, nb), jnp.float32),
        compiler_params=pltpu.CompilerParams(
            dimension_semantics=("arbitrary",)),
    )(xf, yf, zf, rx, ry, rz)


def _make_sc_merge(num_blocks, n_recv):
    n_groups = n_recv // _LANES
    mesh = plsc.VectorSubcoreMesh(core_axis_name="c", subcore_axis_name="s",
                                  num_cores=2, num_subcores=16)

    @functools.partial(
        pl.kernel,
        out_type=jax.ShapeDtypeStruct((n_recv,), jnp.int32),
        mesh=mesh,
        scratch_types=[
            pltpu.VMEM((num_blocks, 1, n_recv), jnp.float32),
            pltpu.VMEM((n_recv,), jnp.int32),
        ],
        compiler_params=pltpu.CompilerParams(needs_layout_passes=False),
    )
    def sc_merge(vals_hbm, wb_hbm, vals_v, wb_v):
        cid = lax.axis_index("c")
        sid = lax.axis_index("s")
        leader = jnp.logical_and(cid == 0, sid == 0)

        @pl.when(leader)
        def _():
            pltpu.sync_copy(vals_hbm, vals_v)
            inf16 = jnp.full((_LANES,), jnp.inf, jnp.float32)
            zero16 = jnp.zeros((_LANES,), jnp.int32)

            def mbody(b, carry):
                bvec = zero16 + b
                new = []
                for g in range(n_groups):
                    cv, ci = carry[2 * g], carry[2 * g + 1]
                    rv = vals_v[b, 0, pl.ds(g * _LANES, _LANES)]
                    m = rv < cv
                    new.append(jnp.where(m, rv, cv))
                    new.append(jnp.where(m, bvec, ci))
                return tuple(new)

            init = tuple(x for _ in range(n_groups) for x in (inf16, zero16))
            res = lax.fori_loop(0, num_blocks, mbody, init)
            for g in range(n_groups):
                wb_v[pl.ds(g * _LANES, _LANES)] = res[2 * g + 1]
            pltpu.sync_copy(wb_v, wb_hbm)

    return sc_merge


def _refine_body(wb_ref, rxb_ref, ryb_ref, rzb_ref,
                 x_ref, y_ref, z_ref, out_ref, *, blk, n_points):
    i = pl.program_id(0)
    wb = wb_ref[i]
    rx = lax.bitcast_convert_type(rxb_ref[i], jnp.float32)
    ry = lax.bitcast_convert_type(ryb_ref[i], jnp.float32)
    rz = lax.bitcast_convert_type(rzb_ref[i], jnp.float32)
    rem = n_points - wb * blk
    ok = lax.broadcasted_iota(jnp.int32, (1, blk), 1) < rem
    x = jnp.where(ok, x_ref[...].reshape(1, blk), 1e9)
    y = jnp.where(ok, y_ref[...].reshape(1, blk), 1e9)
    z = jnp.where(ok, z_ref[...].reshape(1, blk), 1e9)
    dx = x - rx
    dy = y - ry
    dz = z - rz
    d2 = (dx * dx + dy * dy) + dz * dz
    lv = jnp.min(d2, axis=1)
    pos = lax.broadcasted_iota(jnp.int32, d2.shape, 1)
    big = jnp.iinfo(jnp.int32).max
    la = jnp.min(jnp.where(d2 == lv[:, None], pos, big), axis=1)
    g = la + wb * blk
    out_ref[...] = jnp.broadcast_to(g[:, None, None], (1, 1, 128))


def _refine(wb, rxb, ryb, rzb, xf, yf, zf, num_blocks, blk, n_points):
    n_recv = wb.shape[0]
    coord_spec = pl.BlockSpec(
        (blk,), lambda i, wb_ref, *_: (wb_ref[i],))
    return pl.pallas_call(
        functools.partial(_refine_body, blk=blk, n_points=n_points),
        grid_spec=pltpu.PrefetchScalarGridSpec(
            num_scalar_prefetch=4,
            grid=(n_recv,),
            in_specs=[coord_spec, coord_spec, coord_spec],
            out_specs=pl.BlockSpec((1, 1, 128), lambda i, *_: (i, 0, 0)),
        ),
        out_shape=jax.ShapeDtypeStruct((n_recv, 1, 128), jnp.int32),
        compiler_params=pltpu.CompilerParams(
            dimension_semantics=("arbitrary",)),
    )(wb, rxb, ryb, rzb, xf, yf, zf)


def _make_sc_finish(n_recv, n_points):
    n_groups = n_recv // _LANES
    n_tiles = 16
    zchunk = (n_points // (n_tiles * _LANES)) * _LANES
    tail = n_points - zchunk * n_tiles
    n_cl = 3 * n_recv
    n_oh = n_recv + _LANES

    mesh = plsc.VectorSubcoreMesh(core_axis_name="c", subcore_axis_name="s",
                                  num_cores=2, num_subcores=16)

    @functools.partial(
        pl.kernel,
        out_type=[
            jax.ShapeDtypeStruct((n_cl,), jnp.float32),
            jax.ShapeDtypeStruct((n_points,), jnp.float32),
        ],
        mesh=mesh,
        scratch_types=[
            pltpu.VMEM((zchunk,), jnp.float32),
            pltpu.VMEM((n_recv,), jnp.int32),
            pltpu.VMEM((n_recv,), jnp.float32),
            pltpu.VMEM((n_cl,), jnp.float32),
            pltpu.VMEM((n_oh,), jnp.int32),
            pltpu.VMEM((n_oh,), jnp.float32),
            pltpu.SemaphoreType.DMA,
        ],
        compiler_params=pltpu.CompilerParams(needs_layout_passes=False),
    )
    def sc_finish(minidx_hbm, xf_hbm, yf_hbm, zf_hbm,
                  closest_hbm, oh_hbm,
                  zero_v, minidx_v, cl_v, clflat_v, ohpos_v, ones_v, sem):
        cid = lax.axis_index("c")
        sid = lax.axis_index("s")
        active = cid == 0
        leader = jnp.logical_and(active, sid == 0)

        @pl.when(active)
        def _():
            def zbody(i, c):
                zero_v[pl.ds(i * _LANES, _LANES)] = jnp.zeros(
                    (_LANES,), jnp.float32)
                return c
            lax.fori_loop(0, zchunk // _LANES, zbody, 0)
            pltpu.sync_copy(zero_v, oh_hbm.at[pl.ds(sid * zchunk, zchunk)])

        @pl.when(leader)
        def _():
            if tail:
                pltpu.sync_copy(zero_v.at[pl.ds(0, tail)],
                                oh_hbm.at[pl.ds(n_tiles * zchunk, tail)])

        plsc.subcore_barrier()

        @pl.when(leader)
        def _():
            pltpu.sync_copy(minidx_hbm, minidx_v)
            iota16 = lax.iota(jnp.int32, _LANES)
            zero16 = jnp.zeros((_LANES,), jnp.int32)

            base3 = iota16 * 3
            for c, src in enumerate((xf_hbm, yf_hbm, zf_hbm)):
                pltpu.async_copy(src.at[minidx_v], cl_v, sem).wait()
                for g in range(n_groups):
                    chunk = cl_v[pl.ds(g * _LANES, _LANES)]
                    plsc.store_scatter(
                        clflat_v, [base3 + (3 * _LANES * g + c)], chunk)
            pltpu.sync_copy(clflat_v, closest_hbm)

            for g in range(n_groups):
                ohpos_v[pl.ds(g * _LANES, _LANES)] = minidx_v[
                    pl.ds(g * _LANES, _LANES)]
            ohpos_v[pl.ds(n_recv, _LANES)] = zero16
            for t in range(n_oh // _LANES):
                ones_v[pl.ds(t * _LANES, _LANES)] = jnp.ones(
                    (_LANES,), jnp.float32)
            pltpu.async_copy(ones_v, oh_hbm.at[ohpos_v], sem).wait()

    return sc_finish


def kernel(mesh_3D, receiver_pos):
    lx, ly, lz, _ = mesh_3D.shape
    n_points = lx * ly * lz
    n_recv = receiver_pos.shape[0]
    assert n_recv % _LANES == 0 and n_recv > 1

    xf = mesh_3D[..., 0].reshape(-1)
    yf = mesh_3D[..., 1].reshape(-1)
    zf = mesh_3D[..., 2].reshape(-1)
    blk = _BLK
    num_blocks = -(-n_points // blk)

    rx = receiver_pos[:, 0:1]
    ry = receiver_pos[:, 1:2]
    rz = receiver_pos[:, 2:3]

    vals = _stage_a(xf, yf, zf, rx, ry, rz, num_blocks, blk, n_points)

    sc_merge = _make_sc_merge(num_blocks, n_recv)
    wblock = sc_merge(vals)

    rxb = lax.bitcast_convert_type(receiver_pos[:, 0], jnp.int32)
    ryb = lax.bitcast_convert_type(receiver_pos[:, 1], jnp.int32)
    rzb = lax.bitcast_convert_type(receiver_pos[:, 2], jnp.int32)
    refined = _refine(wblock, rxb, ryb, rzb, xf, yf, zf,
                      num_blocks, blk, n_points)
    min_index = refined[:, 0, 0]

    sc_finish = _make_sc_finish(n_recv, n_points)
    closest_flat, one_hot = sc_finish(min_index, xf, yf, zf)

    input_tensor = jnp.concatenate(
        [xf[:, None], yf[:, None], zf[:, None], one_hot[:, None]],
        axis=1).astype(jnp.float32)
    closest_points = closest_flat.reshape(n_recv, 3)
    return (input_tensor, closest_points, min_index)


# speedup vs baseline: 1.0535x; 1.0535x over previous
import functools

import jax
import jax.numpy as jnp
from jax import lax
from jax.experimental import pallas as pl
from jax.experimental.pallas import tpu as pltpu
from jax.experimental.pallas import tpu_sc as plsc

_BLK = 4096
_LANES = 16


def _scan_body(x_ref, y_ref, z_ref, rx_ref, ry_ref, rz_ref,
               vals_ref, args_ref, *, blk, n_points):
    b = pl.program_id(0)
    rem = n_points - b * blk
    ok = lax.broadcasted_iota(jnp.int32, (1, blk), 1) < rem
    x = jnp.where(ok, x_ref[...].reshape(1, blk), 1e9)
    y = jnp.where(ok, y_ref[...].reshape(1, blk), 1e9)
    z = jnp.where(ok, z_ref[...].reshape(1, blk), 1e9)
    dx = x - rx_ref[...]
    dy = y - ry_ref[...]
    dz = z - rz_ref[...]
    d2 = (dx * dx + dy * dy) + dz * dz
    lv = jnp.min(d2, axis=1)
    la = jnp.argmin(d2, axis=1).astype(jnp.int32)
    vals_ref[...] = lv.reshape(1, 1, -1)
    args_ref[...] = (la + b * blk).reshape(1, 1, -1)


def _stage_a(xf, yf, zf, rx, ry, rz, num_blocks, blk, n_points):
    nb = rx.shape[0]
    coord_spec = pl.BlockSpec((blk,), lambda b: (b,))
    recv_spec = pl.BlockSpec((nb, 1), lambda b: (0, 0))
    out_spec = pl.BlockSpec((1, 1, nb), lambda b: (b, 0, 0))
    return pl.pallas_call(
        functools.partial(_scan_body, blk=blk, n_points=n_points),
        grid=(num_blocks,),
        in_specs=[coord_spec, coord_spec, coord_spec,
                  recv_spec, recv_spec, recv_spec],
        out_specs=[out_spec, out_spec],
        out_shape=[
            jax.ShapeDtypeStruct((num_blocks, 1, nb), jnp.float32),
            jax.ShapeDtypeStruct((num_blocks, 1, nb), jnp.int32),
        ],
        compiler_params=pltpu.CompilerParams(
            dimension_semantics=("arbitrary",)),
    )(xf, yf, zf, rx, ry, rz)


def _make_sc_finish(num_blocks, n_recv, n_points):
    n_groups = n_recv // _LANES
    n_tiles = 16
    zchunk = (n_points // (n_tiles * _LANES)) * _LANES
    tail = n_points - zchunk * n_tiles
    n_cl = 3 * n_recv
    n_oh = n_recv + _LANES

    mesh = plsc.VectorSubcoreMesh(core_axis_name="c", subcore_axis_name="s",
                                  num_cores=2, num_subcores=16)

    @functools.partial(
        pl.kernel,
        out_type=[
            jax.ShapeDtypeStruct((n_recv,), jnp.int32),
            jax.ShapeDtypeStruct((n_cl,), jnp.float32),
            jax.ShapeDtypeStruct((n_points,), jnp.float32),
        ],
        mesh=mesh,
        scratch_types=[
            pltpu.VMEM((num_blocks, 1, n_recv), jnp.float32),
            pltpu.VMEM((num_blocks, 1, n_recv), jnp.int32),
            pltpu.VMEM((zchunk,), jnp.float32),
            pltpu.VMEM((n_recv,), jnp.int32),
            pltpu.VMEM((n_recv,), jnp.float32),
            pltpu.VMEM((n_cl,), jnp.float32),
            pltpu.VMEM((n_oh,), jnp.int32),
            pltpu.VMEM((n_oh,), jnp.float32),
            pltpu.SemaphoreType.DMA,
        ],
        compiler_params=pltpu.CompilerParams(needs_layout_passes=False),
    )
    def sc_finish(vals_hbm, args_hbm, xf_hbm, yf_hbm, zf_hbm,
                  minidx_hbm, closest_hbm, oh_hbm,
                  vals_v, args_v, zero_v, minidx_v, cl_v, clflat_v,
                  ohpos_v, ones_v, sem):
        cid = lax.axis_index("c")
        sid = lax.axis_index("s")
        active = cid == 0
        leader = jnp.logical_and(active, sid == 0)

        @pl.when(active)
        def _():
            def zbody(i, c):
                zero_v[pl.ds(i * _LANES, _LANES)] = jnp.zeros(
                    (_LANES,), jnp.float32)
                return c
            lax.fori_loop(0, zchunk // _LANES, zbody, 0)
            pltpu.sync_copy(zero_v, oh_hbm.at[pl.ds(sid * zchunk, zchunk)])

        @pl.when(leader)
        def _():
            if tail:
                pltpu.sync_copy(zero_v.at[pl.ds(0, tail)],
                                oh_hbm.at[pl.ds(n_tiles * zchunk, tail)])

        plsc.subcore_barrier()

        @pl.when(leader)
        def _():
            pltpu.sync_copy(vals_hbm, vals_v)
            pltpu.sync_copy(args_hbm, args_v)

            inf16 = jnp.full((_LANES,), jnp.inf, jnp.float32)
            zero16 = jnp.zeros((_LANES,), jnp.int32)
            iota16 = lax.iota(jnp.int32, _LANES)

            def mbody(b, carry):
                new = []
                for g in range(n_groups):
                    cv, ci = carry[2 * g], carry[2 * g + 1]
                    rv = vals_v[b, 0, pl.ds(g * _LANES, _LANES)]
                    ri = args_v[b, 0, pl.ds(g * _LANES, _LANES)]
                    m = rv < cv
                    new.append(jnp.where(m, rv, cv))
                    new.append(jnp.where(m, ri, ci))
                return tuple(new)

            init = tuple(x for _ in range(n_groups) for x in (inf16, zero16))
            res = lax.fori_loop(0, num_blocks, mbody, init)
            for g in range(n_groups):
                minidx_v[pl.ds(g * _LANES, _LANES)] = res[2 * g + 1]
            pltpu.sync_copy(minidx_v, minidx_hbm)

            base3 = iota16 * 3
            for c, src in enumerate((xf_hbm, yf_hbm, zf_hbm)):
                pltpu.async_copy(src.at[minidx_v], cl_v, sem).wait()
                for g in range(n_groups):
                    chunk = cl_v[pl.ds(g * _LANES, _LANES)]
                    plsc.store_scatter(
                        clflat_v, [base3 + (3 * _LANES * g + c)], chunk)
            pltpu.sync_copy(clflat_v, closest_hbm)

            for g in range(n_groups):
                ohpos_v[pl.ds(g * _LANES, _LANES)] = res[2 * g + 1]
            ohpos_v[pl.ds(n_recv, _LANES)] = zero16
            for t in range(n_oh // _LANES):
                ones_v[pl.ds(t * _LANES, _LANES)] = jnp.ones(
                    (_LANES,), jnp.float32)
            pltpu.async_copy(ones_v, oh_hbm.at[ohpos_v], sem).wait()

    return sc_finish


def kernel(mesh_3D, receiver_pos):
    lx, ly, lz, _ = mesh_3D.shape
    n_points = lx * ly * lz
    n_recv = receiver_pos.shape[0]
    assert n_recv % _LANES == 0 and n_recv > 1

    xf = mesh_3D[..., 0].reshape(-1)
    yf = mesh_3D[..., 1].reshape(-1)
    zf = mesh_3D[..., 2].reshape(-1)
    blk = _BLK
    num_blocks = -(-n_points // blk)

    rx = receiver_pos[:, 0:1]
    ry = receiver_pos[:, 1:2]
    rz = receiver_pos[:, 2:3]

    vals, args = _stage_a(xf, yf, zf, rx, ry, rz, num_blocks, blk, n_points)

    sc_finish = _make_sc_finish(num_blocks, n_recv, n_points)
    min_index, closest_flat, one_hot = sc_finish(vals, args, xf, yf, zf)

    input_tensor = jnp.concatenate(
        [xf[:, None], yf[:, None], zf[:, None], one_hot[:, None]],
        axis=1).astype(jnp.float32)
    closest_points = closest_flat.reshape(n_recv, 3)
    return (input_tensor, closest_points, min_index)


# speedup vs baseline: 1.1811x; 1.1211x over previous
import functools

import jax
import jax.numpy as jnp
from jax import lax
from jax.experimental import pallas as pl
from jax.experimental.pallas import tpu as pltpu
from jax.experimental.pallas import tpu_sc as plsc

_BLK = 8192
_LANES = 16


def _scan_body(x_ref, y_ref, z_ref, rx_ref, ry_ref, rz_ref,
               vals_ref, args_ref, *, blk, n_points):
    b = pl.program_id(0)
    rem = n_points - b * blk
    ok = lax.broadcasted_iota(jnp.int32, (1, blk), 1) < rem
    x = jnp.where(ok, x_ref[...].reshape(1, blk), 1e9)
    y = jnp.where(ok, y_ref[...].reshape(1, blk), 1e9)
    z = jnp.where(ok, z_ref[...].reshape(1, blk), 1e9)
    dx = x - rx_ref[...]
    dy = y - ry_ref[...]
    dz = z - rz_ref[...]
    d2 = (dx * dx + dy * dy) + dz * dz
    lv = jnp.min(d2, axis=1)
    pos = lax.broadcasted_iota(jnp.int32, d2.shape, 1)
    big = jnp.iinfo(jnp.int32).max
    la = jnp.min(jnp.where(d2 == lv[:, None], pos, big), axis=1)
    vals_ref[...] = lv.reshape(1, 1, -1)
    args_ref[...] = (la + b * blk).reshape(1, 1, -1)


def _stage_a(xf, yf, zf, rx, ry, rz, num_blocks, blk, n_points):
    nb = rx.shape[0]
    coord_spec = pl.BlockSpec((blk,), lambda b: (b,))
    recv_spec = pl.BlockSpec((nb, 1), lambda b: (0, 0))
    out_spec = pl.BlockSpec((1, 1, nb), lambda b: (b, 0, 0))
    return pl.pallas_call(
        functools.partial(_scan_body, blk=blk, n_points=n_points),
        grid=(num_blocks,),
        in_specs=[coord_spec, coord_spec, coord_spec,
                  recv_spec, recv_spec, recv_spec],
        out_specs=[out_spec, out_spec],
        out_shape=[
            jax.ShapeDtypeStruct((num_blocks, 1, nb), jnp.float32),
            jax.ShapeDtypeStruct((num_blocks, 1, nb), jnp.int32),
        ],
        compiler_params=pltpu.CompilerParams(
            dimension_semantics=("arbitrary",)),
    )(xf, yf, zf, rx, ry, rz)


def _make_sc_finish(num_blocks, n_recv, n_points):
    n_groups = n_recv // _LANES
    n_tiles = 16
    zchunk = (n_points // (n_tiles * _LANES)) * _LANES
    tail = n_points - zchunk * n_tiles
    n_cl = 3 * n_recv
    n_oh = n_recv + _LANES

    mesh = plsc.VectorSubcoreMesh(core_axis_name="c", subcore_axis_name="s",
                                  num_cores=2, num_subcores=16)

    @functools.partial(
        pl.kernel,
        out_type=[
            jax.ShapeDtypeStruct((n_recv,), jnp.int32),
            jax.ShapeDtypeStruct((n_cl,), jnp.float32),
            jax.ShapeDtypeStruct((n_points,), jnp.float32),
        ],
        mesh=mesh,
        scratch_types=[
            pltpu.VMEM((num_blocks, 1, n_recv), jnp.float32),
            pltpu.VMEM((num_blocks, 1, n_recv), jnp.int32),
            pltpu.VMEM((zchunk,), jnp.float32),
            pltpu.VMEM((n_recv,), jnp.int32),
            pltpu.VMEM((n_recv,), jnp.float32),
            pltpu.VMEM((n_cl,), jnp.float32),
            pltpu.VMEM((n_oh,), jnp.int32),
            pltpu.VMEM((n_oh,), jnp.float32),
            pltpu.SemaphoreType.DMA,
        ],
        compiler_params=pltpu.CompilerParams(needs_layout_passes=False),
    )
    def sc_finish(vals_hbm, args_hbm, xf_hbm, yf_hbm, zf_hbm,
                  minidx_hbm, closest_hbm, oh_hbm,
                  vals_v, args_v, zero_v, minidx_v, cl_v, clflat_v,
                  ohpos_v, ones_v, sem):
        cid = lax.axis_index("c")
        sid = lax.axis_index("s")
        active = cid == 0
        leader = jnp.logical_and(active, sid == 0)

        @pl.when(active)
        def _():
            def zbody(i, c):
                zero_v[pl.ds(i * _LANES, _LANES)] = jnp.zeros(
                    (_LANES,), jnp.float32)
                return c
            lax.fori_loop(0, zchunk // _LANES, zbody, 0)
            pltpu.sync_copy(zero_v, oh_hbm.at[pl.ds(sid * zchunk, zchunk)])

        @pl.when(leader)
        def _():
            if tail:
                pltpu.sync_copy(zero_v.at[pl.ds(0, tail)],
                                oh_hbm.at[pl.ds(n_tiles * zchunk, tail)])

        plsc.subcore_barrier()

        @pl.when(leader)
        def _():
            pltpu.sync_copy(vals_hbm, vals_v)
            pltpu.sync_copy(args_hbm, args_v)

            inf16 = jnp.full((_LANES,), jnp.inf, jnp.float32)
            zero16 = jnp.zeros((_LANES,), jnp.int32)
            iota16 = lax.iota(jnp.int32, _LANES)

            def mbody(b, carry):
                new = []
                for g in range(n_groups):
                    cv, ci = carry[2 * g], carry[2 * g + 1]
                    rv = vals_v[b, 0, pl.ds(g * _LANES, _LANES)]
                    ri = args_v[b, 0, pl.ds(g * _LANES, _LANES)]
                    m = rv < cv
                    new.append(jnp.where(m, rv, cv))
                    new.append(jnp.where(m, ri, ci))
                return tuple(new)

            init = tuple(x for _ in range(n_groups) for x in (inf16, zero16))
            res = lax.fori_loop(0, num_blocks, mbody, init)
            for g in range(n_groups):
                minidx_v[pl.ds(g * _LANES, _LANES)] = res[2 * g + 1]
            pltpu.sync_copy(minidx_v, minidx_hbm)

            base3 = iota16 * 3
            for c, src in enumerate((xf_hbm, yf_hbm, zf_hbm)):
                pltpu.async_copy(src.at[minidx_v], cl_v, sem).wait()
                for g in range(n_groups):
                    chunk = cl_v[pl.ds(g * _LANES, _LANES)]
                    plsc.store_scatter(
                        clflat_v, [base3 + (3 * _LANES * g + c)], chunk)
            pltpu.sync_copy(clflat_v, closest_hbm)

            for g in range(n_groups):
                ohpos_v[pl.ds(g * _LANES, _LANES)] = res[2 * g + 1]
            ohpos_v[pl.ds(n_recv, _LANES)] = zero16
            for t in range(n_oh // _LANES):
                ones_v[pl.ds(t * _LANES, _LANES)] = jnp.ones(
                    (_LANES,), jnp.float32)
            pltpu.async_copy(ones_v, oh_hbm.at[ohpos_v], sem).wait()

    return sc_finish


def kernel(mesh_3D, receiver_pos):
    lx, ly, lz, _ = mesh_3D.shape
    n_points = lx * ly * lz
    n_recv = receiver_pos.shape[0]
    assert n_recv % _LANES == 0 and n_recv > 1

    xf = mesh_3D[..., 0].reshape(-1)
    yf = mesh_3D[..., 1].reshape(-1)
    zf = mesh_3D[..., 2].reshape(-1)
    blk = _BLK
    num_blocks = -(-n_points // blk)

    rx = receiver_pos[:, 0:1]
    ry = receiver_pos[:, 1:2]
    rz = receiver_pos[:, 2:3]

    vals, args = _stage_a(xf, yf, zf, rx, ry, rz, num_blocks, blk, n_points)

    sc_finish = _make_sc_finish(num_blocks, n_recv, n_points)
    min_index, closest_flat, one_hot = sc_finish(vals, args, xf, yf, zf)

    input_tensor = jnp.concatenate(
        [xf[:, None], yf[:, None], zf[:, None], one_hot[:, None]],
        axis=1).astype(jnp.float32)
    closest_points = closest_flat.reshape(n_recv, 3)
    return (input_tensor, closest_points, min_index)


# speedup vs baseline: 1.2495x; 1.0579x over previous
import functools

import jax
import jax.numpy as jnp
from jax import lax
from jax.experimental import pallas as pl
from jax.experimental.pallas import tpu as pltpu
from jax.experimental.pallas import tpu_sc as plsc

_BLK = 16384
_LANES = 16


def _scan_body(x_ref, y_ref, z_ref, rx_ref, ry_ref, rz_ref,
               vals_ref, args_ref, *, blk, n_points):
    b = pl.program_id(0)
    rem = n_points - b * blk
    ok = lax.broadcasted_iota(jnp.int32, (1, blk), 1) < rem
    x = jnp.where(ok, x_ref[...].reshape(1, blk), 1e9)
    y = jnp.where(ok, y_ref[...].reshape(1, blk), 1e9)
    z = jnp.where(ok, z_ref[...].reshape(1, blk), 1e9)
    dx = x - rx_ref[...]
    dy = y - ry_ref[...]
    dz = z - rz_ref[...]
    d2 = (dx * dx + dy * dy) + dz * dz
    lv = jnp.min(d2, axis=1)
    pos = lax.broadcasted_iota(jnp.int32, d2.shape, 1)
    big = jnp.iinfo(jnp.int32).max
    la = jnp.min(jnp.where(d2 == lv[:, None], pos, big), axis=1)
    vals_ref[...] = lv.reshape(1, 1, -1)
    args_ref[...] = (la + b * blk).reshape(1, 1, -1)


def _stage_a(xf, yf, zf, rx, ry, rz, num_blocks, blk, n_points):
    nb = rx.shape[0]
    coord_spec = pl.BlockSpec((blk,), lambda b: (b,))
    recv_spec = pl.BlockSpec((nb, 1), lambda b: (0, 0))
    out_spec = pl.BlockSpec((1, 1, nb), lambda b: (b, 0, 0))
    return pl.pallas_call(
        functools.partial(_scan_body, blk=blk, n_points=n_points),
        grid=(num_blocks,),
        in_specs=[coord_spec, coord_spec, coord_spec,
                  recv_spec, recv_spec, recv_spec],
        out_specs=[out_spec, out_spec],
        out_shape=[
            jax.ShapeDtypeStruct((num_blocks, 1, nb), jnp.float32),
            jax.ShapeDtypeStruct((num_blocks, 1, nb), jnp.int32),
        ],
        compiler_params=pltpu.CompilerParams(
            dimension_semantics=("arbitrary",)),
    )(xf, yf, zf, rx, ry, rz)


def _make_sc_finish(num_blocks, n_recv, n_points):
    n_groups = n_recv // _LANES
    n_tiles = 16
    zchunk = (n_points // (n_tiles * _LANES)) * _LANES
    tail = n_points - zchunk * n_tiles
    n_cl = 3 * n_recv
    n_oh = n_recv + _LANES

    mesh = plsc.VectorSubcoreMesh(core_axis_name="c", subcore_axis_name="s",
                                  num_cores=2, num_subcores=16)

    @functools.partial(
        pl.kernel,
        out_type=[
            jax.ShapeDtypeStruct((n_recv,), jnp.int32),
            jax.ShapeDtypeStruct((n_cl,), jnp.float32),
            jax.ShapeDtypeStruct((n_points,), jnp.float32),
        ],
        mesh=mesh,
        scratch_types=[
            pltpu.VMEM((num_blocks, 1, n_recv), jnp.float32),
            pltpu.VMEM((num_blocks, 1, n_recv), jnp.int32),
            pltpu.VMEM((zchunk,), jnp.float32),
            pltpu.VMEM((n_recv,), jnp.int32),
            pltpu.VMEM((n_recv,), jnp.float32),
            pltpu.VMEM((n_cl,), jnp.float32),
            pltpu.VMEM((n_oh,), jnp.int32),
            pltpu.VMEM((n_oh,), jnp.float32),
            pltpu.SemaphoreType.DMA,
        ],
        compiler_params=pltpu.CompilerParams(needs_layout_passes=False),
    )
    def sc_finish(vals_hbm, args_hbm, xf_hbm, yf_hbm, zf_hbm,
                  minidx_hbm, closest_hbm, oh_hbm,
                  vals_v, args_v, zero_v, minidx_v, cl_v, clflat_v,
                  ohpos_v, ones_v, sem):
        cid = lax.axis_index("c")
        sid = lax.axis_index("s")
        active = cid == 0
        leader = jnp.logical_and(active, sid == 0)

        @pl.when(active)
        def _():
            def zbody(i, c):
                zero_v[pl.ds(i * _LANES, _LANES)] = jnp.zeros(
                    (_LANES,), jnp.float32)
                return c
            lax.fori_loop(0, zchunk // _LANES, zbody, 0)
            pltpu.sync_copy(zero_v, oh_hbm.at[pl.ds(sid * zchunk, zchunk)])

        @pl.when(leader)
        def _():
            if tail:
                pltpu.sync_copy(zero_v.at[pl.ds(0, tail)],
                                oh_hbm.at[pl.ds(n_tiles * zchunk, tail)])

        plsc.subcore_barrier()

        @pl.when(leader)
        def _():
            pltpu.sync_copy(vals_hbm, vals_v)
            pltpu.sync_copy(args_hbm, args_v)

            inf16 = jnp.full((_LANES,), jnp.inf, jnp.float32)
            zero16 = jnp.zeros((_LANES,), jnp.int32)
            iota16 = lax.iota(jnp.int32, _LANES)

            def mbody(b, carry):
                new = []
                for g in range(n_groups):
                    cv, ci = carry[2 * g], carry[2 * g + 1]
                    rv = vals_v[b, 0, pl.ds(g * _LANES, _LANES)]
                    ri = args_v[b, 0, pl.ds(g * _LANES, _LANES)]
                    m = rv < cv
                    new.append(jnp.where(m, rv, cv))
                    new.append(jnp.where(m, ri, ci))
                return tuple(new)

            init = tuple(x for _ in range(n_groups) for x in (inf16, zero16))
            res = lax.fori_loop(0, num_blocks, mbody, init)
            for g in range(n_groups):
                minidx_v[pl.ds(g * _LANES, _LANES)] = res[2 * g + 1]
            pltpu.sync_copy(minidx_v, minidx_hbm)

            base3 = iota16 * 3
            for c, src in enumerate((xf_hbm, yf_hbm, zf_hbm)):
                pltpu.async_copy(src.at[minidx_v], cl_v, sem).wait()
                for g in range(n_groups):
                    chunk = cl_v[pl.ds(g * _LANES, _LANES)]
                    plsc.store_scatter(
                        clflat_v, [base3 + (3 * _LANES * g + c)], chunk)
            pltpu.sync_copy(clflat_v, closest_hbm)

            for g in range(n_groups):
                ohpos_v[pl.ds(g * _LANES, _LANES)] = res[2 * g + 1]
            ohpos_v[pl.ds(n_recv, _LANES)] = zero16
            for t in range(n_oh // _LANES):
                ones_v[pl.ds(t * _LANES, _LANES)] = jnp.ones(
                    (_LANES,), jnp.float32)
            pltpu.async_copy(ones_v, oh_hbm.at[ohpos_v], sem).wait()

    return sc_finish


def kernel(mesh_3D, receiver_pos):
    lx, ly, lz, _ = mesh_3D.shape
    n_points = lx * ly * lz
    n_recv = receiver_pos.shape[0]
    assert n_recv % _LANES == 0 and n_recv > 1

    xf = mesh_3D[..., 0].reshape(-1)
    yf = mesh_3D[..., 1].reshape(-1)
    zf = mesh_3D[..., 2].reshape(-1)
    blk = _BLK
    num_blocks = -(-n_points // blk)

    rx = receiver_pos[:, 0:1]
    ry = receiver_pos[:, 1:2]
    rz = receiver_pos[:, 2:3]

    vals, args = _stage_a(xf, yf, zf, rx, ry, rz, num_blocks, blk, n_points)

    sc_finish = _make_sc_finish(num_blocks, n_recv, n_points)
    min_index, closest_flat, one_hot = sc_finish(vals, args, xf, yf, zf)

    input_tensor = jnp.concatenate(
        [xf[:, None], yf[:, None], zf[:, None], one_hot[:, None]],
        axis=1).astype(jnp.float32)
    closest_points = closest_flat.reshape(n_recv, 3)
    return (input_tensor, closest_points, min_index)


# speedup vs baseline: 1.2709x; 1.0171x over previous
import functools

import jax
import jax.numpy as jnp
from jax import lax
from jax.experimental import pallas as pl
from jax.experimental.pallas import tpu as pltpu
from jax.experimental.pallas import tpu_sc as plsc

_BLK = 32768
_LANES = 16


def _scan_body(x_ref, y_ref, z_ref, rx_ref, ry_ref, rz_ref,
               vals_ref, args_ref, *, blk, n_points):
    b = pl.program_id(0)
    rem = n_points - b * blk
    ok = lax.broadcasted_iota(jnp.int32, (1, blk), 1) < rem
    x = jnp.where(ok, x_ref[...].reshape(1, blk), 1e9)
    y = jnp.where(ok, y_ref[...].reshape(1, blk), 1e9)
    z = jnp.where(ok, z_ref[...].reshape(1, blk), 1e9)
    dx = x - rx_ref[...]
    dy = y - ry_ref[...]
    dz = z - rz_ref[...]
    d2 = (dx * dx + dy * dy) + dz * dz
    lv = jnp.min(d2, axis=1)
    pos = lax.broadcasted_iota(jnp.int32, d2.shape, 1)
    big = jnp.iinfo(jnp.int32).max
    la = jnp.min(jnp.where(d2 == lv[:, None], pos, big), axis=1)
    vals_ref[...] = lv.reshape(1, 1, -1)
    args_ref[...] = (la + b * blk).reshape(1, 1, -1)


def _stage_a(xf, yf, zf, rx, ry, rz, num_blocks, blk, n_points):
    nb = rx.shape[0]
    coord_spec = pl.BlockSpec((blk,), lambda b: (b,))
    recv_spec = pl.BlockSpec((nb, 1), lambda b: (0, 0))
    out_spec = pl.BlockSpec((1, 1, nb), lambda b: (b, 0, 0))
    return pl.pallas_call(
        functools.partial(_scan_body, blk=blk, n_points=n_points),
        grid=(num_blocks,),
        in_specs=[coord_spec, coord_spec, coord_spec,
                  recv_spec, recv_spec, recv_spec],
        out_specs=[out_spec, out_spec],
        out_shape=[
            jax.ShapeDtypeStruct((num_blocks, 1, nb), jnp.float32),
            jax.ShapeDtypeStruct((num_blocks, 1, nb), jnp.int32),
        ],
        compiler_params=pltpu.CompilerParams(
            dimension_semantics=("arbitrary",)),
    )(xf, yf, zf, rx, ry, rz)


def _make_sc_finish(num_blocks, n_recv, n_points):
    n_groups = n_recv // _LANES
    n_tiles = 16
    zchunk = (n_points // (n_tiles * _LANES)) * _LANES
    tail = n_points - zchunk * n_tiles
    n_cl = 3 * n_recv
    n_oh = n_recv + _LANES

    mesh = plsc.VectorSubcoreMesh(core_axis_name="c", subcore_axis_name="s",
                                  num_cores=2, num_subcores=16)

    @functools.partial(
        pl.kernel,
        out_type=[
            jax.ShapeDtypeStruct((n_recv,), jnp.int32),
            jax.ShapeDtypeStruct((n_cl,), jnp.float32),
            jax.ShapeDtypeStruct((n_points,), jnp.float32),
        ],
        mesh=mesh,
        scratch_types=[
            pltpu.VMEM((num_blocks, 1, n_recv), jnp.float32),
            pltpu.VMEM((num_blocks, 1, n_recv), jnp.int32),
            pltpu.VMEM((zchunk,), jnp.float32),
            pltpu.VMEM((n_recv,), jnp.int32),
            pltpu.VMEM((n_recv,), jnp.float32),
            pltpu.VMEM((n_cl,), jnp.float32),
            pltpu.VMEM((n_oh,), jnp.int32),
            pltpu.VMEM((n_oh,), jnp.float32),
            pltpu.SemaphoreType.DMA,
        ],
        compiler_params=pltpu.CompilerParams(needs_layout_passes=False),
    )
    def sc_finish(vals_hbm, args_hbm, xf_hbm, yf_hbm, zf_hbm,
                  minidx_hbm, closest_hbm, oh_hbm,
                  vals_v, args_v, zero_v, minidx_v, cl_v, clflat_v,
                  ohpos_v, ones_v, sem):
        cid = lax.axis_index("c")
        sid = lax.axis_index("s")
        active = cid == 0
        leader = jnp.logical_and(active, sid == 0)

        @pl.when(active)
        def _():
            def zbody(i, c):
                zero_v[pl.ds(i * _LANES, _LANES)] = jnp.zeros(
                    (_LANES,), jnp.float32)
                return c
            lax.fori_loop(0, zchunk // _LANES, zbody, 0)
            pltpu.sync_copy(zero_v, oh_hbm.at[pl.ds(sid * zchunk, zchunk)])

        @pl.when(leader)
        def _():
            if tail:
                pltpu.sync_copy(zero_v.at[pl.ds(0, tail)],
                                oh_hbm.at[pl.ds(n_tiles * zchunk, tail)])

        plsc.subcore_barrier()

        @pl.when(leader)
        def _():
            pltpu.sync_copy(vals_hbm, vals_v)
            pltpu.sync_copy(args_hbm, args_v)

            inf16 = jnp.full((_LANES,), jnp.inf, jnp.float32)
            zero16 = jnp.zeros((_LANES,), jnp.int32)
            iota16 = lax.iota(jnp.int32, _LANES)

            def mbody(b, carry):
                new = []
                for g in range(n_groups):
                    cv, ci = carry[2 * g], carry[2 * g + 1]
                    rv = vals_v[b, 0, pl.ds(g * _LANES, _LANES)]
                    ri = args_v[b, 0, pl.ds(g * _LANES, _LANES)]
                    m = rv < cv
                    new.append(jnp.where(m, rv, cv))
                    new.append(jnp.where(m, ri, ci))
                return tuple(new)

            init = tuple(x for _ in range(n_groups) for x in (inf16, zero16))
            res = lax.fori_loop(0, num_blocks, mbody, init)
            for g in range(n_groups):
                minidx_v[pl.ds(g * _LANES, _LANES)] = res[2 * g + 1]
            pltpu.sync_copy(minidx_v, minidx_hbm)

            base3 = iota16 * 3
            for c, src in enumerate((xf_hbm, yf_hbm, zf_hbm)):
                pltpu.async_copy(src.at[minidx_v], cl_v, sem).wait()
                for g in range(n_groups):
                    chunk = cl_v[pl.ds(g * _LANES, _LANES)]
                    plsc.store_scatter(
                        clflat_v, [base3 + (3 * _LANES * g + c)], chunk)
            pltpu.sync_copy(clflat_v, closest_hbm)

            for g in range(n_groups):
                ohpos_v[pl.ds(g * _LANES, _LANES)] = res[2 * g + 1]
            ohpos_v[pl.ds(n_recv, _LANES)] = zero16
            for t in range(n_oh // _LANES):
                ones_v[pl.ds(t * _LANES, _LANES)] = jnp.ones(
                    (_LANES,), jnp.float32)
            pltpu.async_copy(ones_v, oh_hbm.at[ohpos_v], sem).wait()

    return sc_finish


def kernel(mesh_3D, receiver_pos):
    lx, ly, lz, _ = mesh_3D.shape
    n_points = lx * ly * lz
    n_recv = receiver_pos.shape[0]
    assert n_recv % _LANES == 0 and n_recv > 1

    xf = mesh_3D[..., 0].reshape(-1)
    yf = mesh_3D[..., 1].reshape(-1)
    zf = mesh_3D[..., 2].reshape(-1)
    blk = _BLK
    num_blocks = -(-n_points // blk)

    rx = receiver_pos[:, 0:1]
    ry = receiver_pos[:, 1:2]
    rz = receiver_pos[:, 2:3]

    vals, args = _stage_a(xf, yf, zf, rx, ry, rz, num_blocks, blk, n_points)

    sc_finish = _make_sc_finish(num_blocks, n_recv, n_points)
    min_index, closest_flat, one_hot = sc_finish(vals, args, xf, yf, zf)

    input_tensor = jnp.concatenate(
        [xf[:, None], yf[:, None], zf[:, None], one_hot[:, None]],
        axis=1).astype(jnp.float32)
    closest_points = closest_flat.reshape(n_recv, 3)
    return (input_tensor, closest_points, min_index)
